# Initial kernel scaffold; baseline (speedup 1.0000x reference)
import jax
import jax.numpy as jnp
from jax.experimental import pallas as pl


def kernel(x, edge_ids, edge_attr, W1, b1, W2, b2):
    raise NotImplementedError("write your pallas kernel here")


# speedup vs baseline: 2.2516x; 2.2516x over previous
import functools

import jax
import jax.numpy as jnp
from jax import lax
from jax.experimental import pallas as pl
from jax.experimental.pallas import tpu as pltpu
from jax.experimental.pallas import tpu_sc as plsc

N = 10000
E = 320000
D = 128
ED = 16
AW = 144
L = 16

NC, NS = 2, 16
EPB = 80
NB = E // EPB
NB_CORE = NB // NC
NB_TILE = NB_CORE // NS
RPT = N // NS


def _xa_body(x_ref, w_ref, o_ref):
    o_ref[...] = jnp.dot(x_ref[...], w_ref[...],
                         preferred_element_type=jnp.float32)


def _eb_body(a_ref, w_ref, b_ref, o_ref):
    o_ref[...] = jnp.dot(a_ref[...], w_ref[...],
                         preferred_element_type=jnp.float32) + b_ref[...]


_sc_mesh = plsc.VectorSubcoreMesh(core_axis_name="c", subcore_axis_name="s")


@functools.partial(
    pl.kernel,
    out_type=jax.ShapeDtypeStruct((NC, N, AW), jnp.float32),
    mesh=_sc_mesh,
    compiler_params=pltpu.CompilerParams(use_tc_tiling_on_sc=False),
    scratch_types=[
        pltpu.VMEM((EPB,), jnp.int32),
        pltpu.VMEM((EPB,), jnp.int32),
        pltpu.VMEM((EPB, D), jnp.float32),
        pltpu.VMEM((EPB, D), jnp.float32),
        pltpu.VMEM((EPB, AW), jnp.float32),
        pltpu.VMEM_SHARED((N, AW), jnp.float32),
        pltpu.SemaphoreType.DMA,
        pltpu.SemaphoreType.DMA,
    ],
)
def _sc_scatter(row_hbm, col_hbm, xa_hbm, eb_hbm, zeros_hbm, out_hbm,
                ridx, cidx, gbuf, mbuf, obuf, acc, sem_g, sem_m):
    c = lax.axis_index("c")
    s = lax.axis_index("s")

    pltpu.sync_copy(zeros_hbm, acc.at[pl.ds(s * RPT, RPT)])
    plsc.subcore_barrier()

    ones = jnp.ones((L,), jnp.float32)

    def batch_body(k, carry):
        b = c * NB_CORE + k * NS + s
        ebase = b * EPB
        pltpu.sync_copy(row_hbm.at[pl.ds(ebase, EPB)], ridx)
        pltpu.sync_copy(col_hbm.at[pl.ds(ebase, EPB)], cidx)
        cp_g = pltpu.async_copy(xa_hbm.at[ridx], gbuf, sem_g)
        cp_m = pltpu.async_copy(eb_hbm.at[pl.ds(ebase, EPB)], mbuf, sem_m)
        cp_g.wait()
        cp_m.wait()

        def edge_body(e, c2):
            for j in range(D // L):
                g = gbuf[e, pl.ds(j * L, L)]
                m = mbuf[e, pl.ds(j * L, L)]
                obuf[e, pl.ds(j * L, L)] = jnp.maximum(g + m, 0.0)
            obuf[e, pl.ds(D, L)] = ones
            return c2

        lax.fori_loop(0, EPB, edge_body, 0)
        pltpu.sync_copy(obuf, acc.at[cidx], add=True)
        return carry

    lax.fori_loop(0, NB_TILE, batch_body, 0)

    plsc.subcore_barrier()
    pltpu.sync_copy(acc.at[pl.ds(s * RPT, RPT)],
                    out_hbm.at[c, pl.ds(s * RPT, RPT)])


def _fin_body(acc_ref, x_ref, wa_ref, wb_ref, b_ref, o_ref):
    summed = acc_ref[0, :, :D] + acc_ref[1, :, :D]
    cnt = acc_ref[0, :, D:D + 1] + acc_ref[1, :, D:D + 1]
    mean = summed / jnp.maximum(cnt, 1.0)
    o_ref[...] = jnp.maximum(
        jnp.dot(x_ref[...], wa_ref[...], preferred_element_type=jnp.float32)
        + jnp.dot(mean, wb_ref[...], preferred_element_type=jnp.float32)
        + b_ref[...], 0.0)


def kernel(x, edge_ids, edge_attr, W1, b1, W2, b2):
    row = edge_ids[0].astype(jnp.int32)
    col = edge_ids[1].astype(jnp.int32)
    W1a, W1b = W1[:D], W1[D:]
    W2a, W2b = W2[:D], W2[D:]

    xa = pl.pallas_call(
        _xa_body,
        grid=(10,),
        in_specs=[pl.BlockSpec((N // 10, D), lambda i: (i, 0)),
                  pl.BlockSpec((D, D), lambda i: (0, 0))],
        out_specs=pl.BlockSpec((N // 10, D), lambda i: (i, 0)),
        out_shape=jax.ShapeDtypeStruct((N, D), jnp.float32),
    )(x, W1a)

    eb = pl.pallas_call(
        _eb_body,
        grid=(80,),
        in_specs=[pl.BlockSpec((E // 80, ED), lambda i: (i, 0)),
                  pl.BlockSpec((ED, D), lambda i: (0, 0)),
                  pl.BlockSpec((1, D), lambda i: (0, 0))],
        out_specs=pl.BlockSpec((E // 80, D), lambda i: (i, 0)),
        out_shape=jax.ShapeDtypeStruct((E, D), jnp.float32),
    )(edge_attr, W1b, b1.reshape(1, D))

    zeros = jnp.zeros((RPT, AW), jnp.float32)
    acc = _sc_scatter(row, col, xa, eb, zeros)

    out = pl.pallas_call(
        _fin_body,
        grid=(10,),
        in_specs=[pl.BlockSpec((NC, N // 10, AW), lambda i: (0, i, 0)),
                  pl.BlockSpec((N // 10, D), lambda i: (i, 0)),
                  pl.BlockSpec((D, D), lambda i: (0, 0)),
                  pl.BlockSpec((D, D), lambda i: (0, 0)),
                  pl.BlockSpec((1, D), lambda i: (0, 0))],
        out_specs=pl.BlockSpec((N // 10, D), lambda i: (i, 0)),
        out_shape=jax.ShapeDtypeStruct((N, D), jnp.float32),
    )(acc, x, W2a, W2b, b2.reshape(1, D))
    return out
